# baseline (speedup 1.0000x reference)
import functools

import jax
import jax.numpy as jnp
from jax import lax
from jax.experimental import pallas as pl
from jax.experimental.pallas import tpu as pltpu
from jax.experimental.pallas import tpu_sc as plsc

VOCAB = 1000000
EMB = 32
HID = 128
OUT = 16
BATCH = 16384

NC = 2
NS = 16
NW = NC * NS
B_PER_W = BATCH // NW
K = 16
LAG = 8


def _sc_gather_body(emb_hbm, idx_hbm, out_hbm, idx_v, idx_s, rows_v, sem):
    c = lax.axis_index("c")
    s = lax.axis_index("s")
    wid = s * NC + c
    base = wid * B_PER_W
    pltpu.sync_copy(idx_hbm.at[pl.ds(base, B_PER_W)], idx_v)

    @plsc.parallel_loop(0, B_PER_W // K, 1, unroll=1)
    def _(i):
        off = i * K
        vec = idx_v[pl.ds(off, K)]
        for k in range(K):
            r = jnp.sum(jnp.where(lax.iota(jnp.int32, K) == k, vec, 0))
            idx_s[off + k] = r

    def fire(off):
        for k in range(K):
            pltpu.async_copy(
                emb_hbm.at[pl.ds(idx_s[off + k], 1), :],
                rows_v.at[pl.ds(off + k, 1), :],
                sem,
            )

    def drain(off):
        pltpu.make_async_copy(
            emb_hbm.at[pl.ds(0, K)], rows_v.at[pl.ds(off, K)], sem
        ).wait()

    ngrp = B_PER_W // K
    for g in range(LAG):
        fire(g * K)

    def grp(i, carry):
        fire((i + LAG) * K)
        drain(i * K)
        return carry

    lax.fori_loop(0, ngrp - LAG, grp, 0)
    for g in range(ngrp - LAG, ngrp):
        drain(g * K)
    pltpu.sync_copy(rows_v, out_hbm.at[pl.ds(base, B_PER_W)])


@jax.jit
def _sc_gather(emb, idx):
    mesh = plsc.VectorSubcoreMesh(core_axis_name="c", subcore_axis_name="s")
    k = functools.partial(
        pl.kernel,
        mesh=mesh,
        out_type=jax.ShapeDtypeStruct((BATCH, EMB), jnp.float32),
        scratch_types=[
            pltpu.VMEM((B_PER_W,), jnp.int32),
            pltpu.SMEM((B_PER_W,), jnp.int32),
            pltpu.VMEM((B_PER_W, EMB), jnp.float32),
            pltpu.SemaphoreType.DMA,
        ],
        compiler_params=pltpu.CompilerParams(needs_layout_passes=False),
    )(_sc_gather_body)
    return k(emb, idx)


def _mlp_body(h_ref, w1_ref, b1_ref, w2_ref, b2_ref, w3_ref, b3_ref, o_ref):
    h = h_ref[...]
    z = jnp.dot(h, w1_ref[...], preferred_element_type=jnp.float32)
    z = jnp.maximum(z + b1_ref[...], 0.0)
    z = jnp.dot(z, w2_ref[...], preferred_element_type=jnp.float32)
    z = jnp.maximum(z + b2_ref[...], 0.0)
    z = jnp.dot(z, w3_ref[...], preferred_element_type=jnp.float32)
    o_ref[...] = z + b3_ref[...]


BLK = 16384


@jax.jit
def _tc_mlp(h, W1, b1, W2, b2, W3, b3):
    grid = (BATCH // BLK,)
    full = lambda shape: pl.BlockSpec(shape, lambda i: (0, 0))
    return pl.pallas_call(
        _mlp_body,
        grid=grid,
        in_specs=[
            pl.BlockSpec((BLK, EMB), lambda i: (i, 0)),
            full((EMB, HID)),
            full((1, HID)),
            full((HID, HID)),
            full((1, HID)),
            full((HID, OUT)),
            full((1, OUT)),
        ],
        out_specs=pl.BlockSpec((BLK, OUT), lambda i: (i, 0)),
        out_shape=jax.ShapeDtypeStruct((BATCH, OUT), jnp.float32),
    )(h, W1, b1, W2, b2, W3, b3)


def kernel(x, emb, W1, b1, W2, b2, W3, b3):
    rows = _sc_gather(emb, x.astype(jnp.int32))
    return _tc_mlp(
        rows,
        W1,
        b1.reshape(1, HID),
        W2,
        b2.reshape(1, HID),
        W3,
        b3.reshape(1, OUT),
    )


# speedup vs baseline: 1.0059x; 1.0059x over previous
import functools

import jax
import jax.numpy as jnp
from jax import lax
from jax.experimental import pallas as pl
from jax.experimental.pallas import tpu as pltpu
from jax.experimental.pallas import tpu_sc as plsc

VOCAB = 1000000
EMB = 32
HID = 128
OUT = 16
BATCH = 16384

NC = 2
NS = 16
NW = NC * NS
B_PER_W = BATCH // NW
K = 16
LAG = 8


def _sc_gather_body(emb_hbm, idx_hbm, out_hbm, idx_v, idx_s, rows_v, sem):
    c = lax.axis_index("c")
    s = lax.axis_index("s")
    wid = s * NC + c
    base = wid * B_PER_W
    pltpu.sync_copy(idx_hbm.at[pl.ds(base, B_PER_W)], idx_v)

    @plsc.parallel_loop(0, B_PER_W // K, 1, unroll=1)
    def _(i):
        off = i * K
        vec = idx_v[pl.ds(off, K)]
        for k in range(K):
            r = jnp.sum(jnp.where(lax.iota(jnp.int32, K) == k, vec, 0))
            idx_s[off + k] = r

    def fire(off):
        for k in range(K):
            pltpu.async_copy(
                emb_hbm.at[pl.ds(idx_s[off + k], 1), :],
                rows_v.at[pl.ds(off + k, 1), :],
                sem,
            )

    def drain(off):
        pltpu.make_async_copy(
            emb_hbm.at[pl.ds(0, K)], rows_v.at[pl.ds(off, K)], sem
        ).wait()

    ngrp = B_PER_W // K
    for g in range(LAG):
        fire(g * K)

    def grp(i, carry):
        fire((i + LAG) * K)
        drain(i * K)
        return carry

    lax.fori_loop(0, ngrp - LAG, grp, 0)
    for g in range(ngrp - LAG, ngrp):
        drain(g * K)
    pltpu.sync_copy(rows_v, out_hbm.at[pl.ds(base, B_PER_W)])


@jax.jit
def _sc_gather(emb, idx):
    mesh = plsc.VectorSubcoreMesh(core_axis_name="c", subcore_axis_name="s")
    k = functools.partial(
        pl.kernel,
        mesh=mesh,
        out_type=jax.ShapeDtypeStruct((BATCH, EMB), jnp.float32),
        scratch_types=[
            pltpu.VMEM((B_PER_W,), jnp.int32),
            pltpu.SMEM((B_PER_W,), jnp.int32),
            pltpu.VMEM((B_PER_W, EMB), jnp.float32),
            pltpu.SemaphoreType.DMA,
        ],
        compiler_params=pltpu.CompilerParams(needs_layout_passes=False),
    )(_sc_gather_body)
    return k(emb, idx)


def _mlp_body(h_ref, w1_ref, b1_ref, w2_ref, b2_ref, w3_ref, b3_ref, o_ref):
    h = h_ref[...]
    z = jnp.dot(h, w1_ref[...], preferred_element_type=jnp.float32)
    z = jnp.maximum(z + b1_ref[...], 0.0)
    z = jnp.dot(z, w2_ref[...], preferred_element_type=jnp.float32)
    z = jnp.maximum(z + b2_ref[...], 0.0)
    z = jnp.dot(z, w3_ref[...], preferred_element_type=jnp.float32)
    o_ref[...] = z + b3_ref[...]


BLK = 8192


@jax.jit
def _tc_mlp(h, W1, b1, W2, b2, W3, b3):
    grid = (BATCH // BLK,)
    full = lambda shape: pl.BlockSpec(shape, lambda i: (0, 0))
    return pl.pallas_call(
        _mlp_body,
        grid=grid,
        in_specs=[
            pl.BlockSpec((BLK, EMB), lambda i: (i, 0)),
            full((EMB, HID)),
            full((1, HID)),
            full((HID, HID)),
            full((1, HID)),
            full((HID, OUT)),
            full((1, OUT)),
        ],
        out_specs=pl.BlockSpec((BLK, OUT), lambda i: (i, 0)),
        out_shape=jax.ShapeDtypeStruct((BATCH, OUT), jnp.float32),
    )(h, W1, b1, W2, b2, W3, b3)


def kernel(x, emb, W1, b1, W2, b2, W3, b3):
    rows = _sc_gather(emb, x.astype(jnp.int32))
    return _tc_mlp(
        rows,
        W1,
        b1.reshape(1, HID),
        W2,
        b2.reshape(1, HID),
        W3,
        b3.reshape(1, OUT),
    )
